# baseline (speedup 1.0000x reference)
import dataclasses
import functools

import jax
import jax.numpy as jnp
from jax import lax
from jax.experimental import pallas as pl
from jax.experimental.pallas import tpu as pltpu
from jax.experimental.pallas import tpu_sc as plsc

_SC_COMPILER_PARAMS = pltpu.CompilerParams()
if "needs_layout_passes" in pltpu.CompilerParams.__dataclass_fields__:
    _SC_COMPILER_PARAMS = dataclasses.replace(
        _SC_COMPILER_PARAMS, needs_layout_passes=False)

NC = 2
NS = 16
NW = NC * NS
CH = 64
GI = 32
LANES = 16


def _sc_aggregate(x, src3, dst3, zeros_f, zeros_d, r_pad, n_groups):
    n, f = x.shape
    rps = r_pad // NS

    mesh = plsc.VectorSubcoreMesh(core_axis_name="c", subcore_axis_name="s")

    @functools.partial(
        pl.kernel,
        out_type=(
            jax.ShapeDtypeStruct((NC * r_pad, f), jnp.float32),
            jax.ShapeDtypeStruct((NW, r_pad), jnp.float32),
        ),
        mesh=mesh,
        compiler_params=_SC_COMPILER_PARAMS,
        scratch_types=[
            pltpu.VMEM((GI, CH), jnp.int32),
            pltpu.VMEM((GI, CH), jnp.int32),
            pltpu.VMEM((CH, f), jnp.float32),
            pltpu.VMEM((CH, f), jnp.float32),
            pltpu.VMEM((r_pad,), jnp.float32),
            pltpu.SemaphoreType.DMA,
            pltpu.SemaphoreType.DMA,
            pltpu.SemaphoreType.DMA,
            pltpu.SemaphoreType.DMA,
            pltpu.VMEM_SHARED((r_pad, f), jnp.float32),
        ],
    )
    def sc_kernel(x_hbm, src_hbm, dst_hbm, zf_hbm, zd_hbm,
                  agg_out, deg_out,
                  srcg_v, dstg_v, rows_a, rows_b, cnt_v,
                  gsem_a, gsem_b, ssem_a, ssem_b, agg_sh):
        c = lax.axis_index("c")
        s = lax.axis_index("s")
        wid = s * NC + c
        base = s * rps
        out_base = c * r_pad + base
        bufs = (rows_a, rows_b)
        gsems = (gsem_a, gsem_b)
        ssems = (ssem_a, ssem_b)
        ones16 = jnp.full((LANES,), 1.0, jnp.float32)

        pltpu.sync_copy(zf_hbm, agg_sh.at[pl.ds(base, rps)])
        pltpu.sync_copy(zd_hbm, cnt_v)
        plsc.subcore_barrier()

        @pl.when(c == 1)
        def _run_groups():
          @pl.loop(0, n_groups)
          def _(g):
            row = s * n_groups + g
            pltpu.sync_copy(src_hbm.at[row], srcg_v)
            pltpu.sync_copy(dst_hbm.at[row], dstg_v)

            gather_d = [None] * GI
            scatter_d = [None] * GI
            gather_d[0] = pltpu.async_copy(
                x_hbm.at[srcg_v.at[0]], bufs[0], gsems[0])
            for r in range(GI):
                b = r % 2
                if r + 1 < GI:
                    if r >= 1:
                        scatter_d[r - 1].wait()
                    gather_d[r + 1] = pltpu.async_copy(
                        x_hbm.at[srcg_v.at[r + 1]], bufs[1 - b],
                        gsems[1 - b])
                for k in range(CH // LANES):
                    idx16 = dstg_v[r, pl.ds(k * LANES, LANES)]
                    plsc.addupdate_scatter(cnt_v, [idx16], ones16)
                gather_d[r].wait()
                scatter_d[r] = pltpu.async_copy(
                    bufs[b], agg_sh.at[dstg_v.at[r]], ssems[b], add=True)
            scatter_d[GI - 2].wait()
            scatter_d[GI - 1].wait()

        plsc.subcore_barrier()
        pltpu.sync_copy(agg_sh.at[pl.ds(base, rps)],
                        agg_out.at[pl.ds(out_base, rps)])
        pltpu.sync_copy(cnt_v, deg_out.at[wid])

    return sc_kernel(x, src3, dst3, zeros_f, zeros_d)


def _tc_finish(x, agg_a, agg_b, deg_t, W, b):
    n, f = x.shape
    o = W.shape[0]
    nw = deg_t.shape[1]
    br = 2000

    def body(x_ref, aa_ref, ab_ref, d_ref, w_ref, b_ref, o_ref):
        agg = aa_ref[...] + ab_ref[...]
        deg_col = jnp.sum(d_ref[...], axis=1, keepdims=True)
        h_n = agg / jnp.maximum(deg_col, 1.0)
        w1 = w_ref[:, :f]
        w2 = w_ref[:, f:]
        acc = lax.dot_general(x_ref[...], w1, (((1,), (1,)), ((), ())),
                              preferred_element_type=jnp.float32)
        acc = acc + lax.dot_general(h_n, w2, (((1,), (1,)), ((), ())),
                                    preferred_element_type=jnp.float32)
        o_ref[...] = acc + b_ref[...]

    return pl.pallas_call(
        body,
        grid=(n // br,),
        in_specs=[
            pl.BlockSpec((br, f), lambda i: (i, 0)),
            pl.BlockSpec((br, f), lambda i: (i, 0)),
            pl.BlockSpec((br, f), lambda i: (i, 0)),
            pl.BlockSpec((br, nw), lambda i: (i, 0)),
            pl.BlockSpec((f, 2 * f), lambda i: (0, 0)),
            pl.BlockSpec((1, o), lambda i: (0, 0)),
        ],
        out_specs=pl.BlockSpec((br, o), lambda i: (i, 0)),
        out_shape=jax.ShapeDtypeStruct((n, o), jnp.float32),
    )(x, agg_a, agg_b, deg_t, W, b.reshape(1, o))


def kernel(x, edge_index, W, b):
    n, f = x.shape
    e = edge_index.shape[1]

    n_groups = -(-e // (NS * GI * CH))
    e_pad = n_groups * NS * GI * CH
    r_pad = -(-(n + 1) // (NS * 8)) * (NS * 8)

    src = edge_index[0].astype(jnp.int32)
    dst = edge_index[1].astype(jnp.int32)
    pad = e_pad - e
    src = jnp.concatenate([src, jnp.zeros((pad,), jnp.int32)])
    dst = jnp.concatenate([dst, jnp.full((pad,), n, jnp.int32)])
    src3 = src.reshape(NS * n_groups, GI, CH)
    dst3 = dst.reshape(NS * n_groups, GI, CH)

    rps = r_pad // NS
    zeros_f = jnp.zeros((rps, f), jnp.float32)
    zeros_d = jnp.zeros((r_pad,), jnp.float32)

    agg, deg = _sc_aggregate(x, src3, dst3, zeros_f, zeros_d,
                             r_pad, n_groups)
    agg_a = agg[:n]
    agg_b = agg[r_pad:r_pad + n]
    deg_t = deg.T[:n]
    return _tc_finish(x, agg_a, agg_b, deg_t, W, b)


# speedup vs baseline: 1.0658x; 1.0658x over previous
import dataclasses
import functools

import jax
import jax.numpy as jnp
from jax import lax
from jax.experimental import pallas as pl
from jax.experimental.pallas import tpu as pltpu
from jax.experimental.pallas import tpu_sc as plsc

_SC_COMPILER_PARAMS = pltpu.CompilerParams()
if "needs_layout_passes" in pltpu.CompilerParams.__dataclass_fields__:
    _SC_COMPILER_PARAMS = dataclasses.replace(
        _SC_COMPILER_PARAMS, needs_layout_passes=False)

NC = 2
NS = 16
NW = NC * NS
CH = 64
GI = 32
LANES = 16


def _sc_aggregate(x, src3, dst3, zeros_f, zeros_d, r_pad, n_groups):
    n, f = x.shape
    rps = r_pad // NS

    mesh = plsc.VectorSubcoreMesh(core_axis_name="c", subcore_axis_name="s")

    @functools.partial(
        pl.kernel,
        out_type=(
            jax.ShapeDtypeStruct((NC * r_pad, f), jnp.float32),
            jax.ShapeDtypeStruct((NW, r_pad), jnp.float32),
        ),
        mesh=mesh,
        compiler_params=_SC_COMPILER_PARAMS,
        scratch_types=[
            pltpu.VMEM((GI, CH), jnp.int32),
            pltpu.VMEM((GI, CH), jnp.int32),
            pltpu.VMEM((CH, f), jnp.float32),
            pltpu.VMEM((CH, f), jnp.float32),
            pltpu.VMEM((r_pad,), jnp.float32),
            pltpu.SemaphoreType.DMA,
            pltpu.SemaphoreType.DMA,
            pltpu.SemaphoreType.DMA,
            pltpu.SemaphoreType.DMA,
            pltpu.VMEM_SHARED((r_pad, f), jnp.float32),
        ],
    )
    def sc_kernel(x_hbm, src_hbm, dst_hbm, zf_hbm, zd_hbm,
                  agg_out, deg_out,
                  srcg_v, dstg_v, rows_a, rows_b, cnt_v,
                  gsem_a, gsem_b, ssem_a, ssem_b, agg_sh):
        c = lax.axis_index("c")
        s = lax.axis_index("s")
        wid = s * NC + c
        base = s * rps
        out_base = c * r_pad + base
        bufs = (rows_a, rows_b)
        gsems = (gsem_a, gsem_b)
        ssems = (ssem_a, ssem_b)
        ones16 = jnp.full((LANES,), 1.0, jnp.float32)

        pltpu.sync_copy(zf_hbm, agg_sh.at[pl.ds(base, rps)])
        pltpu.sync_copy(zd_hbm, cnt_v)
        plsc.subcore_barrier()

        a_g = max(1, (2 * n_groups) // 5)
        b_g = 2 * n_groups - a_g
        n_my = lax.select(c == 0, a_g, b_g)
        base_row = lax.select(c == 0, s * a_g, NS * a_g + s * b_g)

        @pl.loop(0, n_my)
        def _(g):
            row = base_row + g
            pltpu.sync_copy(src_hbm.at[row], srcg_v)
            pltpu.sync_copy(dst_hbm.at[row], dstg_v)

            gather_d = [None] * GI
            scatter_d = [None] * GI
            gather_d[0] = pltpu.async_copy(
                x_hbm.at[srcg_v.at[0]], bufs[0], gsems[0])
            for r in range(GI):
                b = r % 2
                if r + 1 < GI:
                    if r >= 1:
                        scatter_d[r - 1].wait()
                    gather_d[r + 1] = pltpu.async_copy(
                        x_hbm.at[srcg_v.at[r + 1]], bufs[1 - b],
                        gsems[1 - b])
                for k in range(CH // LANES):
                    idx16 = dstg_v[r, pl.ds(k * LANES, LANES)]
                    plsc.addupdate_scatter(cnt_v, [idx16], ones16)
                gather_d[r].wait()
                scatter_d[r] = pltpu.async_copy(
                    bufs[b], agg_sh.at[dstg_v.at[r]], ssems[b], add=True)
            scatter_d[GI - 2].wait()
            scatter_d[GI - 1].wait()

        plsc.subcore_barrier()
        pltpu.sync_copy(agg_sh.at[pl.ds(base, rps)],
                        agg_out.at[pl.ds(out_base, rps)])
        pltpu.sync_copy(cnt_v, deg_out.at[wid])

    return sc_kernel(x, src3, dst3, zeros_f, zeros_d)


def _tc_finish(x, agg_a, agg_b, deg_t, W, b):
    n, f = x.shape
    o = W.shape[0]
    nw = deg_t.shape[1]
    br = 2000

    def body(x_ref, aa_ref, ab_ref, d_ref, w_ref, b_ref, o_ref):
        agg = aa_ref[...] + ab_ref[...]
        deg_col = jnp.sum(d_ref[...], axis=1, keepdims=True)
        h_n = agg / jnp.maximum(deg_col, 1.0)
        w1 = w_ref[:, :f]
        w2 = w_ref[:, f:]
        acc = lax.dot_general(x_ref[...], w1, (((1,), (1,)), ((), ())),
                              preferred_element_type=jnp.float32)
        acc = acc + lax.dot_general(h_n, w2, (((1,), (1,)), ((), ())),
                                    preferred_element_type=jnp.float32)
        o_ref[...] = acc + b_ref[...]

    return pl.pallas_call(
        body,
        grid=(n // br,),
        in_specs=[
            pl.BlockSpec((br, f), lambda i: (i, 0)),
            pl.BlockSpec((br, f), lambda i: (i, 0)),
            pl.BlockSpec((br, f), lambda i: (i, 0)),
            pl.BlockSpec((br, nw), lambda i: (i, 0)),
            pl.BlockSpec((f, 2 * f), lambda i: (0, 0)),
            pl.BlockSpec((1, o), lambda i: (0, 0)),
        ],
        out_specs=pl.BlockSpec((br, o), lambda i: (i, 0)),
        out_shape=jax.ShapeDtypeStruct((n, o), jnp.float32),
    )(x, agg_a, agg_b, deg_t, W, b.reshape(1, o))


def kernel(x, edge_index, W, b):
    n, f = x.shape
    e = edge_index.shape[1]

    n_groups = -(-e // (NW * GI * CH))
    e_pad = n_groups * NW * GI * CH
    r_pad = -(-(n + 1) // (NS * 8)) * (NS * 8)

    src = edge_index[0].astype(jnp.int32)
    dst = edge_index[1].astype(jnp.int32)
    pad = e_pad - e
    src = jnp.concatenate([src, jnp.zeros((pad,), jnp.int32)])
    dst = jnp.concatenate([dst, jnp.full((pad,), n, jnp.int32)])
    src3 = src.reshape(NW * n_groups, GI, CH)
    dst3 = dst.reshape(NW * n_groups, GI, CH)

    rps = r_pad // NS
    zeros_f = jnp.zeros((rps, f), jnp.float32)
    zeros_d = jnp.zeros((r_pad,), jnp.float32)

    agg, deg = _sc_aggregate(x, src3, dst3, zeros_f, zeros_d,
                             r_pad, n_groups)
    agg_a = agg[:n]
    agg_b = agg[r_pad:r_pad + n]
    deg_t = deg.T[:n]
    return _tc_finish(x, agg_a, agg_b, deg_t, W, b)


# speedup vs baseline: 1.1786x; 1.1058x over previous
import dataclasses
import functools

import jax
import jax.numpy as jnp
from jax import lax
from jax.experimental import pallas as pl
from jax.experimental.pallas import tpu as pltpu
from jax.experimental.pallas import tpu_sc as plsc

_SC_COMPILER_PARAMS = pltpu.CompilerParams()
if "needs_layout_passes" in pltpu.CompilerParams.__dataclass_fields__:
    _SC_COMPILER_PARAMS = dataclasses.replace(
        _SC_COMPILER_PARAMS, needs_layout_passes=False)

NC = 2
NS = 16
NW = NC * NS
CH = 64
GI = 32
LANES = 16


def _sc_aggregate(x, src3, dst3, zeros_f, zeros_d, r_pad, n_groups):
    n, f = x.shape
    rps = r_pad // NS

    mesh = plsc.VectorSubcoreMesh(core_axis_name="c", subcore_axis_name="s")

    @functools.partial(
        pl.kernel,
        out_type=(
            jax.ShapeDtypeStruct((NC * r_pad, f), jnp.float32),
            jax.ShapeDtypeStruct((NW, r_pad), jnp.float32),
        ),
        mesh=mesh,
        compiler_params=_SC_COMPILER_PARAMS,
        scratch_types=[
            pltpu.VMEM((GI, CH), jnp.int32),
            pltpu.VMEM((GI, CH), jnp.int32),
            pltpu.VMEM((CH, f), jnp.float32),
            pltpu.VMEM((CH, f), jnp.float32),
            pltpu.VMEM((r_pad,), jnp.float32),
            pltpu.SemaphoreType.DMA,
            pltpu.SemaphoreType.DMA,
            pltpu.SemaphoreType.DMA,
            pltpu.SemaphoreType.DMA,
            pltpu.VMEM_SHARED((r_pad, f), jnp.float32),
        ],
    )
    def sc_kernel(x_hbm, src_hbm, dst_hbm, zf_hbm, zd_hbm,
                  agg_out, deg_out,
                  srcg_v, dstg_v, rows_a, rows_b, cnt_v,
                  gsem_a, gsem_b, ssem_a, ssem_b, agg_sh):
        c = lax.axis_index("c")
        s = lax.axis_index("s")
        wid = s * NC + c
        base = s * rps
        out_base = c * r_pad + base
        bufs = (rows_a, rows_b)
        gsems = (gsem_a, gsem_b)
        ssems = (ssem_a, ssem_b)
        ones16 = jnp.full((LANES,), 1.0, jnp.float32)

        pltpu.sync_copy(zf_hbm, agg_sh.at[pl.ds(base, rps)])
        pltpu.sync_copy(zd_hbm, cnt_v)
        plsc.subcore_barrier()

        @pl.loop(0, n_groups)
        def _(g):
            row = wid * n_groups + g
            pltpu.sync_copy(src_hbm.at[row], srcg_v)
            pltpu.sync_copy(dst_hbm.at[row], dstg_v)

            gather_d = [None] * GI
            scatter_d = [None] * GI
            gather_d[0] = pltpu.async_copy(
                x_hbm.at[srcg_v.at[0]], bufs[0], gsems[0])
            for r in range(GI):
                b = r % 2
                if r + 1 < GI:
                    if r >= 1:
                        scatter_d[r - 1].wait()
                    gather_d[r + 1] = pltpu.async_copy(
                        x_hbm.at[srcg_v.at[r + 1]], bufs[1 - b],
                        gsems[1 - b])
                for k in range(CH // LANES):
                    idx16 = dstg_v[r, pl.ds(k * LANES, LANES)]
                    plsc.addupdate_scatter(cnt_v, [idx16], ones16)
                gather_d[r].wait()
                scatter_d[r] = pltpu.async_copy(
                    bufs[b], agg_sh.at[dstg_v.at[r]], ssems[b], add=True)
            scatter_d[GI - 2].wait()
            scatter_d[GI - 1].wait()

        plsc.subcore_barrier()
        pltpu.sync_copy(agg_sh.at[pl.ds(base, rps)],
                        agg_out.at[pl.ds(out_base, rps)])
        pltpu.sync_copy(cnt_v, deg_out.at[wid])

    return sc_kernel(x, src3, dst3, zeros_f, zeros_d)


def _tc_y1(x, W1, b):
    n, f = x.shape
    o = W1.shape[0]
    br = 2000

    def body(x_ref, w_ref, b_ref, o_ref):
        o_ref[...] = lax.dot_general(
            x_ref[...], w_ref[...], (((1,), (1,)), ((), ())),
            preferred_element_type=jnp.float32) + b_ref[...]

    return pl.pallas_call(
        body,
        grid=(n // br,),
        in_specs=[
            pl.BlockSpec((br, f), lambda i: (i, 0)),
            pl.BlockSpec((o, f), lambda i: (0, 0)),
            pl.BlockSpec((1, o), lambda i: (0, 0)),
        ],
        out_specs=pl.BlockSpec((br, o), lambda i: (i, 0)),
        out_shape=jax.ShapeDtypeStruct((n, o), jnp.float32),
    )(x, W1, b.reshape(1, o))


def _tc_finish(y1, agg_a, agg_b, deg_t, W2):
    n, f = agg_a.shape
    o = W2.shape[0]
    nw = deg_t.shape[1]
    br = 2000

    def body(y_ref, aa_ref, ab_ref, d_ref, w_ref, o_ref):
        agg = aa_ref[...] + ab_ref[...]
        deg_col = jnp.sum(d_ref[...], axis=1, keepdims=True)
        h_n = agg / jnp.maximum(deg_col, 1.0)
        o_ref[...] = y_ref[...] + lax.dot_general(
            h_n, w_ref[...], (((1,), (1,)), ((), ())),
            preferred_element_type=jnp.float32)

    return pl.pallas_call(
        body,
        grid=(n // br,),
        in_specs=[
            pl.BlockSpec((br, o), lambda i: (i, 0)),
            pl.BlockSpec((br, f), lambda i: (i, 0)),
            pl.BlockSpec((br, f), lambda i: (i, 0)),
            pl.BlockSpec((br, nw), lambda i: (i, 0)),
            pl.BlockSpec((o, f), lambda i: (0, 0)),
        ],
        out_specs=pl.BlockSpec((br, o), lambda i: (i, 0)),
        out_shape=jax.ShapeDtypeStruct((n, o), jnp.float32),
    )(y1, agg_a, agg_b, deg_t, W2)


def kernel(x, edge_index, W, b):
    n, f = x.shape
    e = edge_index.shape[1]

    n_groups = -(-e // (NW * GI * CH))
    e_pad = n_groups * NW * GI * CH
    r_pad = -(-(n + 1) // (NS * 8)) * (NS * 8)

    src = edge_index[0].astype(jnp.int32)
    dst = edge_index[1].astype(jnp.int32)
    pad = e_pad - e
    src = jnp.concatenate([src, jnp.zeros((pad,), jnp.int32)])
    dst = jnp.concatenate([dst, jnp.full((pad,), n, jnp.int32)])
    src3 = src.reshape(NW * n_groups, GI, CH)
    dst3 = dst.reshape(NW * n_groups, GI, CH)

    rps = r_pad // NS
    zeros_f = jnp.zeros((rps, f), jnp.float32)
    zeros_d = jnp.zeros((r_pad,), jnp.float32)

    y1 = _tc_y1(x, W[:, :f], b)
    agg, deg = _sc_aggregate(x, src3, dst3, zeros_f, zeros_d,
                             r_pad, n_groups)
    agg_a = agg[:n]
    agg_b = agg[r_pad:r_pad + n]
    deg_t = deg.T[:n]
    return _tc_finish(y1, agg_a, agg_b, deg_t, W[:, f:])


# speedup vs baseline: 1.1846x; 1.0051x over previous
import dataclasses
import functools

import jax
import jax.numpy as jnp
from jax import lax
from jax.experimental import pallas as pl
from jax.experimental.pallas import tpu as pltpu
from jax.experimental.pallas import tpu_sc as plsc

_SC_COMPILER_PARAMS = pltpu.CompilerParams()
if "needs_layout_passes" in pltpu.CompilerParams.__dataclass_fields__:
    _SC_COMPILER_PARAMS = dataclasses.replace(
        _SC_COMPILER_PARAMS, needs_layout_passes=False)

NC = 2
NS = 16
NW = NC * NS
CH = 64
GI = 32
LANES = 16


def _sc_aggregate(x, src3, dst3, zeros_f, zeros_d, r_pad, n_groups):
    n, f = x.shape
    rps = r_pad // NS

    mesh = plsc.VectorSubcoreMesh(core_axis_name="c", subcore_axis_name="s")

    @functools.partial(
        pl.kernel,
        out_type=(
            jax.ShapeDtypeStruct((NC * r_pad, f), jnp.float32),
            jax.ShapeDtypeStruct((NW, r_pad), jnp.float32),
        ),
        mesh=mesh,
        compiler_params=_SC_COMPILER_PARAMS,
        scratch_types=[
            pltpu.VMEM((GI, CH), jnp.int32),
            pltpu.VMEM((GI, CH), jnp.int32),
            pltpu.VMEM((CH, f), jnp.float32),
            pltpu.VMEM((CH, f), jnp.float32),
            pltpu.VMEM((r_pad,), jnp.float32),
            pltpu.SemaphoreType.DMA,
            pltpu.SemaphoreType.DMA,
            pltpu.SemaphoreType.DMA,
            pltpu.SemaphoreType.DMA,
            pltpu.VMEM_SHARED((r_pad, f), jnp.float32),
        ],
    )
    def sc_kernel(x_hbm, src_hbm, dst_hbm, zf_hbm, zd_hbm,
                  agg_out, deg_out,
                  srcg_v, dstg_v, rows_a, rows_b, cnt_v,
                  gsem_a, gsem_b, ssem_a, ssem_b, agg_sh):
        c = lax.axis_index("c")
        s = lax.axis_index("s")
        wid = s * NC + c
        base = s * rps
        out_base = c * r_pad + base
        bufs = (rows_a, rows_b)
        gsems = (gsem_a, gsem_b)
        ssems = (ssem_a, ssem_b)
        ones16 = jnp.full((LANES,), 1.0, jnp.float32)

        pltpu.sync_copy(zf_hbm, agg_sh.at[pl.ds(base, rps)])
        pltpu.sync_copy(zd_hbm, cnt_v)
        plsc.subcore_barrier()

        @pl.loop(0, n_groups)
        def _(g):
            row = wid * n_groups + g
            pltpu.sync_copy(src_hbm.at[row], srcg_v)
            pltpu.sync_copy(dst_hbm.at[row], dstg_v)

            gather_d = [None] * GI
            scatter_d = [None] * GI
            gather_d[0] = pltpu.async_copy(
                x_hbm.at[srcg_v.at[0]], bufs[0], gsems[0])
            for r in range(GI):
                b = r % 2
                if r + 1 < GI:
                    if r >= 1:
                        scatter_d[r - 1].wait()
                    gather_d[r + 1] = pltpu.async_copy(
                        x_hbm.at[srcg_v.at[r + 1]], bufs[1 - b],
                        gsems[1 - b])
                for k in range(CH // LANES):
                    idx16 = dstg_v[r, pl.ds(k * LANES, LANES)]
                    plsc.addupdate_scatter(cnt_v, [idx16], ones16)
                gather_d[r].wait()
                scatter_d[r] = pltpu.async_copy(
                    bufs[b], agg_sh.at[dstg_v.at[r]], ssems[b], add=True)
            scatter_d[GI - 2].wait()
            scatter_d[GI - 1].wait()

        plsc.subcore_barrier()
        pltpu.sync_copy(agg_sh.at[pl.ds(base, rps)],
                        agg_out.at[pl.ds(out_base, rps)])
        pltpu.sync_copy(cnt_v, deg_out.at[wid])

    return sc_kernel(x, src3, dst3, zeros_f, zeros_d)


def _tc_finish(x, agg_a, agg_b, deg_t, W, b):
    n, f = x.shape
    o = W.shape[0]
    nw = deg_t.shape[1]
    br = 2000

    def body(x_ref, aa_ref, ab_ref, d_ref, w_ref, b_ref, o_ref):
        agg = aa_ref[...] + ab_ref[...]
        deg_col = jnp.sum(d_ref[...], axis=1, keepdims=True)
        h_n = agg / jnp.maximum(deg_col, 1.0)
        w1 = w_ref[:, :f]
        w2 = w_ref[:, f:]
        acc = lax.dot_general(x_ref[...], w1, (((1,), (1,)), ((), ())),
                              preferred_element_type=jnp.float32)
        acc = acc + lax.dot_general(h_n, w2, (((1,), (1,)), ((), ())),
                                    preferred_element_type=jnp.float32)
        o_ref[...] = acc + b_ref[...]

    return pl.pallas_call(
        body,
        grid=(n // br,),
        in_specs=[
            pl.BlockSpec((br, f), lambda i: (i, 0)),
            pl.BlockSpec((br, f), lambda i: (i, 0)),
            pl.BlockSpec((br, f), lambda i: (i, 0)),
            pl.BlockSpec((br, nw), lambda i: (i, 0)),
            pl.BlockSpec((f, 2 * f), lambda i: (0, 0)),
            pl.BlockSpec((1, o), lambda i: (0, 0)),
        ],
        out_specs=pl.BlockSpec((br, o), lambda i: (i, 0)),
        out_shape=jax.ShapeDtypeStruct((n, o), jnp.float32),
    )(x, agg_a, agg_b, deg_t, W, b.reshape(1, o))


def kernel(x, edge_index, W, b):
    n, f = x.shape
    e = edge_index.shape[1]

    n_groups = -(-e // (NW * GI * CH))
    e_pad = n_groups * NW * GI * CH
    r_pad = -(-(n + 1) // (NS * 8)) * (NS * 8)

    src = edge_index[0].astype(jnp.int32)
    dst = edge_index[1].astype(jnp.int32)
    pad = e_pad - e
    src = jnp.concatenate([src, jnp.zeros((pad,), jnp.int32)])
    dst = jnp.concatenate([dst, jnp.full((pad,), n, jnp.int32)])
    src3 = src.reshape(NW * n_groups, GI, CH)
    dst3 = dst.reshape(NW * n_groups, GI, CH)

    rps = r_pad // NS
    zeros_f = jnp.zeros((rps, f), jnp.float32)
    zeros_d = jnp.zeros((r_pad,), jnp.float32)

    agg, deg = _sc_aggregate(x, src3, dst3, zeros_f, zeros_d,
                             r_pad, n_groups)
    agg_a = agg[:n]
    agg_b = agg[r_pad:r_pad + n]
    deg_t = deg.T[:n]
    return _tc_finish(x, agg_a, agg_b, deg_t, W, b)


# speedup vs baseline: 1.1918x; 1.0061x over previous
import dataclasses
import functools

import jax
import jax.numpy as jnp
from jax import lax
from jax.experimental import pallas as pl
from jax.experimental.pallas import tpu as pltpu
from jax.experimental.pallas import tpu_sc as plsc

_SC_COMPILER_PARAMS = pltpu.CompilerParams()
if "needs_layout_passes" in pltpu.CompilerParams.__dataclass_fields__:
    _SC_COMPILER_PARAMS = dataclasses.replace(
        _SC_COMPILER_PARAMS, needs_layout_passes=False)

NC = 2
NS = 16
NW = NC * NS
CH = 64
GI = 40
LANES = 16


def _sc_aggregate(x, src3, dst3, zeros_f, zeros_d, r_pad, n_groups):
    n, f = x.shape
    rps = r_pad // NS

    mesh = plsc.VectorSubcoreMesh(core_axis_name="c", subcore_axis_name="s")

    @functools.partial(
        pl.kernel,
        out_type=(
            jax.ShapeDtypeStruct((NC * r_pad, f), jnp.float32),
            jax.ShapeDtypeStruct((NW, r_pad), jnp.float32),
        ),
        mesh=mesh,
        compiler_params=_SC_COMPILER_PARAMS,
        scratch_types=[
            pltpu.VMEM((GI, CH), jnp.int32),
            pltpu.VMEM((GI, CH), jnp.int32),
            pltpu.VMEM((CH, f), jnp.float32),
            pltpu.VMEM((CH, f), jnp.float32),
            pltpu.VMEM((r_pad,), jnp.float32),
            pltpu.SemaphoreType.DMA,
            pltpu.SemaphoreType.DMA,
            pltpu.SemaphoreType.DMA,
            pltpu.SemaphoreType.DMA,
            pltpu.VMEM_SHARED((r_pad, f), jnp.float32),
        ],
    )
    def sc_kernel(x_hbm, src_hbm, dst_hbm, zf_hbm, zd_hbm,
                  agg_out, deg_out,
                  srcg_v, dstg_v, rows_a, rows_b, cnt_v,
                  gsem_a, gsem_b, ssem_a, ssem_b, agg_sh):
        c = lax.axis_index("c")
        s = lax.axis_index("s")
        wid = s * NC + c
        base = s * rps
        out_base = c * r_pad + base
        bufs = (rows_a, rows_b)
        gsems = (gsem_a, gsem_b)
        ssems = (ssem_a, ssem_b)
        ones16 = jnp.full((LANES,), 1.0, jnp.float32)

        pltpu.sync_copy(zf_hbm, agg_sh.at[pl.ds(base, rps)])
        pltpu.sync_copy(zd_hbm, cnt_v)
        plsc.subcore_barrier()

        @pl.loop(0, n_groups)
        def _(g):
            row = wid * n_groups + g
            pltpu.sync_copy(src_hbm.at[row], srcg_v)
            pltpu.sync_copy(dst_hbm.at[row], dstg_v)

            gather_d = [None] * GI
            scatter_d = [None] * GI
            gather_d[0] = pltpu.async_copy(
                x_hbm.at[srcg_v.at[0]], bufs[0], gsems[0])
            for r in range(GI):
                b = r % 2
                if r + 1 < GI:
                    if r >= 1:
                        scatter_d[r - 1].wait()
                    gather_d[r + 1] = pltpu.async_copy(
                        x_hbm.at[srcg_v.at[r + 1]], bufs[1 - b],
                        gsems[1 - b])
                for k in range(CH // LANES):
                    idx16 = dstg_v[r, pl.ds(k * LANES, LANES)]
                    plsc.addupdate_scatter(cnt_v, [idx16], ones16)
                gather_d[r].wait()
                scatter_d[r] = pltpu.async_copy(
                    bufs[b], agg_sh.at[dstg_v.at[r]], ssems[b], add=True)
            scatter_d[GI - 2].wait()
            scatter_d[GI - 1].wait()

        plsc.subcore_barrier()
        pltpu.sync_copy(agg_sh.at[pl.ds(base, rps)],
                        agg_out.at[pl.ds(out_base, rps)])
        pltpu.sync_copy(cnt_v, deg_out.at[wid])

    return sc_kernel(x, src3, dst3, zeros_f, zeros_d)


def _tc_finish(x, agg_a, agg_b, deg_t, W, b):
    n, f = x.shape
    o = W.shape[0]
    nw = deg_t.shape[1]
    br = 2000

    def body(x_ref, aa_ref, ab_ref, d_ref, w_ref, b_ref, o_ref):
        agg = aa_ref[...] + ab_ref[...]
        deg_col = jnp.sum(d_ref[...], axis=1, keepdims=True)
        h_n = agg / jnp.maximum(deg_col, 1.0)
        w1 = w_ref[:, :f]
        w2 = w_ref[:, f:]
        acc = lax.dot_general(x_ref[...], w1, (((1,), (1,)), ((), ())),
                              preferred_element_type=jnp.float32)
        acc = acc + lax.dot_general(h_n, w2, (((1,), (1,)), ((), ())),
                                    preferred_element_type=jnp.float32)
        o_ref[...] = acc + b_ref[...]

    return pl.pallas_call(
        body,
        grid=(n // br,),
        in_specs=[
            pl.BlockSpec((br, f), lambda i: (i, 0)),
            pl.BlockSpec((br, f), lambda i: (i, 0)),
            pl.BlockSpec((br, f), lambda i: (i, 0)),
            pl.BlockSpec((br, nw), lambda i: (i, 0)),
            pl.BlockSpec((f, 2 * f), lambda i: (0, 0)),
            pl.BlockSpec((1, o), lambda i: (0, 0)),
        ],
        out_specs=pl.BlockSpec((br, o), lambda i: (i, 0)),
        out_shape=jax.ShapeDtypeStruct((n, o), jnp.float32),
    )(x, agg_a, agg_b, deg_t, W, b.reshape(1, o))


def kernel(x, edge_index, W, b):
    n, f = x.shape
    e = edge_index.shape[1]

    n_groups = -(-e // (NW * GI * CH))
    e_pad = n_groups * NW * GI * CH
    r_pad = -(-(n + 1) // (NS * 8)) * (NS * 8)

    src = edge_index[0].astype(jnp.int32)
    dst = edge_index[1].astype(jnp.int32)
    pad = e_pad - e
    src = jnp.concatenate([src, jnp.zeros((pad,), jnp.int32)])
    dst = jnp.concatenate([dst, jnp.full((pad,), n, jnp.int32)])
    src3 = src.reshape(NW * n_groups, GI, CH)
    dst3 = dst.reshape(NW * n_groups, GI, CH)

    rps = r_pad // NS
    zeros_f = jnp.zeros((rps, f), jnp.float32)
    zeros_d = jnp.zeros((r_pad,), jnp.float32)

    agg, deg = _sc_aggregate(x, src3, dst3, zeros_f, zeros_d,
                             r_pad, n_groups)
    agg_a = agg[:n]
    agg_b = agg[r_pad:r_pad + n]
    deg_t = deg.T[:n]
    return _tc_finish(x, agg_a, agg_b, deg_t, W, b)
